# Initial kernel scaffold; baseline (speedup 1.0000x reference)
import jax
import jax.numpy as jnp
from jax.experimental import pallas as pl


def kernel(x, H_vals, H_rows, H_cols):
    raise NotImplementedError("write your pallas kernel here")


# speedup vs baseline: 80.0660x; 80.0660x over previous
import functools

import jax
import jax.numpy as jnp
from jax import lax
from jax.experimental import pallas as pl
from jax.experimental.pallas import tpu as pltpu
from jax.experimental.pallas import tpu_sc as plsc

M = 92160
N = 512 ** 2
B = 4
NNZ = M * 32
SCALE = 0.82 * 0.0322

NC = 2
NS = 16
NW = NC * NS
PER_W = NNZ // NW
K = 4608
CHUNKS = PER_W // K
Q = K // 16
MH = M // 2
ACC_PER_TILE = MH // NS


def _sc_body(x2_hbm, vals_hbm, rows_hbm, cols_hbm, out_hbm,
             cols_v, colh_v, rows_v, rowh_v, vals_v, xg_v, ct_v, acc_sh):
    c = lax.axis_index("c")
    s = lax.axis_index("s")
    start = (c * NS + s) * PER_W

    ii = lax.iota(jnp.int32, 16)
    zf = jnp.zeros((16,), jnp.float32)
    ir2 = lax.shift_right_logical(ii, 3)
    ic8 = lax.bitwise_and(ii, 7)

    def zb(q, _):
        plsc.store_scatter(ct_v, [q * 2 + ir2, ic8], zf)
        return 0
    lax.fori_loop(0, K // 2, zb, 0)

    r0 = s * ACC_PER_TILE
    pltpu.sync_copy(ct_v.at[pl.ds(0, ACC_PER_TILE)],
                    acc_sh.at[pl.ds(r0, ACC_PER_TILE)])
    plsc.subcore_barrier()

    jvs = [jnp.full((16,), j, jnp.int32) for j in range(B)]

    def chunk_body(i, _):
        base = start + i * K
        pltpu.sync_copy(cols_hbm.at[pl.ds(base, K)], cols_v)
        pltpu.sync_copy(vals_hbm.at[pl.ds(base, K)], vals_v)
        pltpu.sync_copy(rows_hbm.at[pl.ds(base, K)], rows_v)

        def half_body(q, _):
            b16 = q * 16
            colh_v[pl.ds(b16, 16)] = lax.shift_right_logical(
                cols_v[pl.ds(b16, 16)], 1)
            rowh_v[pl.ds(b16, 16)] = lax.shift_right_logical(
                rows_v[pl.ds(b16, 16)], 1)
            return 0
        lax.fori_loop(0, Q, half_body, 0)

        pltpu.sync_copy(x2_hbm.at[colh_v], xg_v)

        def group_body(q, _):
            b16 = q * 16
            e16 = ii + b16
            v16 = vals_v[pl.ds(b16, 16)]
            par4 = lax.shift_left(
                lax.bitwise_and(cols_v[pl.ds(b16, 16)], 1), 2)
            pr4 = lax.shift_left(
                lax.bitwise_and(rows_v[pl.ds(b16, 16)], 1), 2)
            prc4 = lax.bitwise_xor(pr4, 4)
            for j in range(B):
                xf = plsc.load_gather(xg_v, [e16, par4 + jvs[j]])
                plsc.store_scatter(ct_v, [e16, pr4 + jvs[j]], v16 * xf)
                plsc.store_scatter(ct_v, [e16, prc4 + jvs[j]], zf)
            return 0
        lax.fori_loop(0, Q, group_body, 0)

        pltpu.sync_copy(ct_v, acc_sh.at[rowh_v], add=True)
        return 0

    lax.fori_loop(0, CHUNKS, chunk_body, 0)
    plsc.subcore_barrier()
    pltpu.sync_copy(acc_sh.at[pl.ds(r0, ACC_PER_TILE)],
                    out_hbm.at[c, pl.ds(r0, ACC_PER_TILE)])


_sc_project = functools.partial(
    pl.kernel,
    out_type=jax.ShapeDtypeStruct((NC, MH, 8), jnp.float32),
    mesh=plsc.VectorSubcoreMesh(core_axis_name="c", subcore_axis_name="s"),
    compiler_params=pltpu.CompilerParams(
        needs_layout_passes=False, use_tc_tiling_on_sc=False),
    scratch_types=[
        pltpu.VMEM((K,), jnp.int32),
        pltpu.VMEM((K,), jnp.int32),
        pltpu.VMEM((K,), jnp.int32),
        pltpu.VMEM((K,), jnp.int32),
        pltpu.VMEM((K,), jnp.float32),
        pltpu.VMEM((K, 8), jnp.float32),
        pltpu.VMEM((K, 8), jnp.float32),
        pltpu.VMEM_SHARED((MH, 8), jnp.float32),
    ],
)(_sc_body)

_LANES = M * B // 128


def _combine_body(p_ref, o_ref):
    o_ref[...] = SCALE * (p_ref[0] + p_ref[1])


_combine = pl.pallas_call(
    _combine_body,
    out_shape=jax.ShapeDtypeStruct((_LANES, 128), jnp.float32),
)


@jax.jit
def kernel(x, H_vals, H_rows, H_cols):
    x2 = x.reshape(N // 2, 2 * B)
    parts = _sc_project(x2, H_vals, H_rows, H_cols)
    g = _combine(parts.reshape(NC, _LANES, 128))
    return g.reshape(M, B)[None, None]


# speedup vs baseline: 103.6243x; 1.2942x over previous
import functools

import jax
import jax.numpy as jnp
from jax import lax
from jax.experimental import pallas as pl
from jax.experimental.pallas import tpu as pltpu
from jax.experimental.pallas import tpu_sc as plsc

M = 92160
N = 512 ** 2
B = 4
NNZ = M * 32
SCALE = 0.82 * 0.0322

NC = 2
NS = 16
NW = NC * NS
PER_W = NNZ // NW
K = 4608
CHUNKS = PER_W // K
Q = K // 16
MH = M // 2
ACC_PER_TILE = MH // NS


def _sc_body(x2_hbm, vals_hbm, rows_hbm, cols_hbm, out_hbm,
             cols_v, colh_v, rows_v, rowh_v, vals_v, xg_v, ct_v, acc_sh):
    c = lax.axis_index("c")
    s = lax.axis_index("s")
    start = (c * NS + s) * PER_W

    ii = lax.iota(jnp.int32, 16)
    zf = jnp.zeros((16,), jnp.float32)
    ir2 = lax.shift_right_logical(ii, 3)
    ic8 = lax.bitwise_and(ii, 7)

    @plsc.parallel_loop(0, K // 2, unroll=8)
    def _(q):
        plsc.store_scatter(ct_v, [q * 2 + ir2, ic8], zf)

    r0 = s * ACC_PER_TILE
    pltpu.sync_copy(ct_v.at[pl.ds(0, ACC_PER_TILE)],
                    acc_sh.at[pl.ds(r0, ACC_PER_TILE)])
    plsc.subcore_barrier()

    jvs = [jnp.full((16,), j, jnp.int32) for j in range(B)]

    def chunk_body(i, _):
        base = start + i * K
        pltpu.sync_copy(cols_hbm.at[pl.ds(base, K)], cols_v)
        pltpu.sync_copy(vals_hbm.at[pl.ds(base, K)], vals_v)
        pltpu.sync_copy(rows_hbm.at[pl.ds(base, K)], rows_v)

        @plsc.parallel_loop(0, Q, unroll=8)
        def _(q):
            b16 = q * 16
            colh_v[pl.ds(b16, 16)] = lax.shift_right_logical(
                cols_v[pl.ds(b16, 16)], 1)
            rowh_v[pl.ds(b16, 16)] = lax.shift_right_logical(
                rows_v[pl.ds(b16, 16)], 1)

        pltpu.sync_copy(x2_hbm.at[colh_v], xg_v)

        @plsc.parallel_loop(0, Q, unroll=4)
        def _(q):
            b16 = q * 16
            e16 = ii + b16
            v16 = vals_v[pl.ds(b16, 16)]
            par4 = lax.shift_left(
                lax.bitwise_and(cols_v[pl.ds(b16, 16)], 1), 2)
            pr4 = lax.shift_left(
                lax.bitwise_and(rows_v[pl.ds(b16, 16)], 1), 2)
            prc4 = lax.bitwise_xor(pr4, 4)
            for j in range(B):
                xf = plsc.load_gather(xg_v, [e16, par4 + jvs[j]])
                plsc.store_scatter(ct_v, [e16, pr4 + jvs[j]], v16 * xf)
                plsc.store_scatter(ct_v, [e16, prc4 + jvs[j]], zf)

        pltpu.sync_copy(ct_v, acc_sh.at[rowh_v], add=True)
        return 0

    lax.fori_loop(0, CHUNKS, chunk_body, 0)
    plsc.subcore_barrier()
    pltpu.sync_copy(acc_sh.at[pl.ds(r0, ACC_PER_TILE)],
                    out_hbm.at[c, pl.ds(r0, ACC_PER_TILE)])


_sc_project = functools.partial(
    pl.kernel,
    out_type=jax.ShapeDtypeStruct((NC, MH, 8), jnp.float32),
    mesh=plsc.VectorSubcoreMesh(core_axis_name="c", subcore_axis_name="s"),
    compiler_params=pltpu.CompilerParams(
        needs_layout_passes=False, use_tc_tiling_on_sc=False),
    scratch_types=[
        pltpu.VMEM((K,), jnp.int32),
        pltpu.VMEM((K,), jnp.int32),
        pltpu.VMEM((K,), jnp.int32),
        pltpu.VMEM((K,), jnp.int32),
        pltpu.VMEM((K,), jnp.float32),
        pltpu.VMEM((K, 8), jnp.float32),
        pltpu.VMEM((K, 8), jnp.float32),
        pltpu.VMEM_SHARED((MH, 8), jnp.float32),
    ],
)(_sc_body)

_LANES = M * B // 128


def _combine_body(p_ref, o_ref):
    o_ref[...] = SCALE * (p_ref[0] + p_ref[1])


_combine = pl.pallas_call(
    _combine_body,
    out_shape=jax.ShapeDtypeStruct((_LANES, 128), jnp.float32),
)


@jax.jit
def kernel(x, H_vals, H_rows, H_cols):
    x2 = x.reshape(N // 2, 2 * B)
    parts = _sc_project(x2, H_vals, H_rows, H_cols)
    g = _combine(parts.reshape(NC, _LANES, 128))
    return g.reshape(M, B)[None, None]


# speedup vs baseline: 145.1764x; 1.4010x over previous
import functools

import jax
import jax.numpy as jnp
from jax import lax
from jax.experimental import pallas as pl
from jax.experimental.pallas import tpu as pltpu
from jax.experimental.pallas import tpu_sc as plsc

M = 92160
N = 512 ** 2
B = 4
NNZ = M * 32
SCALE = 0.82 * 0.0322

NC = 2
NS = 16
NW = NC * NS
PER_W = NNZ // NW
K = 4608
CHUNKS = PER_W // K
Q = K // 16
MH = M // 2
ACC_PER_TILE = MH // NS

RW = 8192
RROUNDS = (N * B) // (NW * RW)
RG = RW // 16

_mesh = plsc.VectorSubcoreMesh(core_axis_name="c", subcore_axis_name="s")
_cp = pltpu.CompilerParams(needs_layout_passes=False, use_tc_tiling_on_sc=False)


def _repack_body(xp_hbm, x2_hbm, xr_v, xw_v):
    c = lax.axis_index("c")
    s = lax.axis_index("s")
    w = c * NS + s

    ii = lax.iota(jnp.int32, 16)
    ir2 = lax.shift_right_logical(ii, 3)
    ic8 = lax.bitwise_and(ii, 7)
    rconst = lax.bitwise_and(ii, 3) * 128 + 2 * ir2 + \
        lax.shift_right_logical(lax.bitwise_and(ii, 7), 2)

    def rep_round(p, _):
        fb = (w * RROUNDS + p) * RW
        pltpu.sync_copy(xp_hbm.at[pl.ds(fb, RW)], xr_v)

        @plsc.parallel_loop(0, RG, unroll=8)
        def _(g):
            binidx = (g // 32) * 512 + (4 * g) % 128
            v16 = plsc.load_gather(xr_v, [rconst + binidx])
            plsc.store_scatter(xw_v, [2 * g + ir2, ic8], v16)

        pltpu.sync_copy(xw_v, x2_hbm.at[pl.ds((w * RROUNDS + p) * (RW // 8),
                                              RW // 8)])
        return 0
    lax.fori_loop(0, RROUNDS, rep_round, 0)


_repack = functools.partial(
    pl.kernel,
    out_type=jax.ShapeDtypeStruct((N // 2, 8), jnp.float32),
    mesh=_mesh,
    compiler_params=_cp,
    scratch_types=[
        pltpu.VMEM((RW,), jnp.float32),
        pltpu.VMEM((RW // 8, 8), jnp.float32),
    ],
)(_repack_body)


def _sc_body(x2_hbm, vals_hbm, rows_hbm, cols_hbm, out_hbm,
             cols_v, colh_v, rows_v, rowh_v, vals_v, xg_v, ct_v, acc_sh):
    c = lax.axis_index("c")
    s = lax.axis_index("s")
    start = (c * NS + s) * PER_W

    ii = lax.iota(jnp.int32, 16)
    zf = jnp.zeros((16,), jnp.float32)
    ir2 = lax.shift_right_logical(ii, 3)
    ic8 = lax.bitwise_and(ii, 7)

    @plsc.parallel_loop(0, K // 2, unroll=8)
    def _(q):
        plsc.store_scatter(ct_v, [q * 2 + ir2, ic8], zf)

    r0 = s * ACC_PER_TILE
    pltpu.sync_copy(ct_v.at[pl.ds(0, ACC_PER_TILE)],
                    acc_sh.at[pl.ds(r0, ACC_PER_TILE)])
    plsc.subcore_barrier()

    jvs = [jnp.full((16,), j, jnp.int32) for j in range(B)]

    def chunk_body(i, _):
        base = start + i * K
        pltpu.sync_copy(cols_hbm.at[pl.ds(base, K)], cols_v)
        pltpu.sync_copy(vals_hbm.at[pl.ds(base, K)], vals_v)
        pltpu.sync_copy(rows_hbm.at[pl.ds(base, K)], rows_v)

        @plsc.parallel_loop(0, Q, unroll=8)
        def _(q):
            b16 = q * 16
            colh_v[pl.ds(b16, 16)] = lax.shift_right_logical(
                cols_v[pl.ds(b16, 16)], 1)
            rowh_v[pl.ds(b16, 16)] = lax.shift_right_logical(
                rows_v[pl.ds(b16, 16)], 1)

        pltpu.sync_copy(x2_hbm.at[colh_v], xg_v)

        @plsc.parallel_loop(0, Q, unroll=4)
        def _(q):
            b16 = q * 16
            e16 = ii + b16
            v16 = vals_v[pl.ds(b16, 16)]
            par4 = lax.shift_left(
                lax.bitwise_and(cols_v[pl.ds(b16, 16)], 1), 2)
            pr4 = lax.shift_left(
                lax.bitwise_and(rows_v[pl.ds(b16, 16)], 1), 2)
            prc4 = lax.bitwise_xor(pr4, 4)
            for j in range(B):
                xf = plsc.load_gather(xg_v, [e16, par4 + jvs[j]])
                plsc.store_scatter(ct_v, [e16, pr4 + jvs[j]], v16 * xf)
                plsc.store_scatter(ct_v, [e16, prc4 + jvs[j]], zf)

        pltpu.sync_copy(ct_v, acc_sh.at[rowh_v], add=True)
        return 0

    lax.fori_loop(0, CHUNKS, chunk_body, 0)
    plsc.subcore_barrier()
    pltpu.sync_copy(acc_sh.at[pl.ds(r0, ACC_PER_TILE)],
                    out_hbm.at[c, pl.ds(r0, ACC_PER_TILE)])


_sc_project = functools.partial(
    pl.kernel,
    out_type=jax.ShapeDtypeStruct((NC, MH, 8), jnp.float32),
    mesh=_mesh,
    compiler_params=_cp,
    scratch_types=[
        pltpu.VMEM((K,), jnp.int32),
        pltpu.VMEM((K,), jnp.int32),
        pltpu.VMEM((K,), jnp.int32),
        pltpu.VMEM((K,), jnp.int32),
        pltpu.VMEM((K,), jnp.float32),
        pltpu.VMEM((K, 8), jnp.float32),
        pltpu.VMEM((K, 8), jnp.float32),
        pltpu.VMEM_SHARED((MH, 8), jnp.float32),
    ],
)(_sc_body)

_LANES = M * B // 128


def _combine_body(p_ref, o_ref):
    o_ref[...] = SCALE * (p_ref[0] + p_ref[1])


_combine = pl.pallas_call(
    _combine_body,
    out_shape=jax.ShapeDtypeStruct((_LANES, 128), jnp.float32),
)


@jax.jit
def kernel(x, H_vals, H_rows, H_cols):
    xp = x.T.reshape(B, N // 128, 128).transpose(1, 0, 2).reshape(N * B)
    x2 = _repack(xp)
    parts = _sc_project(x2, H_vals, H_rows, H_cols)
    g = _combine(parts.reshape(NC, _LANES, 128))
    return g.reshape(M, B)[None, None]


# speedup vs baseline: 179.7239x; 1.2380x over previous
import functools

import jax
import jax.numpy as jnp
from jax import lax
from jax.experimental import pallas as pl
from jax.experimental.pallas import tpu as pltpu
from jax.experimental.pallas import tpu_sc as plsc

M = 92160
N = 512 ** 2
B = 4
NNZ = M * 32
SCALE = 0.82 * 0.0322

NC = 2
NS = 16
NW = NC * NS
PER_W = NNZ // NW
K = 4608
CHUNKS = PER_W // K
Q = K // 16
MH = M // 2
ACC_PER_TILE = MH // NS

RW = 8192
RROUNDS = (N * B) // (NW * RW)
RG = RW // 16

_mesh = plsc.VectorSubcoreMesh(core_axis_name="c", subcore_axis_name="s")
_cp = pltpu.CompilerParams(needs_layout_passes=False, use_tc_tiling_on_sc=False)


def _repack_body(xp_hbm, x2_hbm, xr_v, xw_v):
    c = lax.axis_index("c")
    s = lax.axis_index("s")
    w = c * NS + s

    ii = lax.iota(jnp.int32, 16)
    ir2 = lax.shift_right_logical(ii, 3)
    ic8 = lax.bitwise_and(ii, 7)
    rconst = lax.bitwise_and(ii, 3) * 128 + 2 * ir2 + \
        lax.shift_right_logical(lax.bitwise_and(ii, 7), 2)

    def rep_round(p, _):
        fb = (w * RROUNDS + p) * RW
        pltpu.sync_copy(xp_hbm.at[pl.ds(fb, RW)], xr_v)

        @plsc.parallel_loop(0, RG, unroll=8)
        def _(g):
            binidx = (g // 32) * 512 + (4 * g) % 128
            v16 = plsc.load_gather(xr_v, [rconst + binidx])
            plsc.store_scatter(xw_v, [2 * g + ir2, ic8], v16)

        pltpu.sync_copy(xw_v, x2_hbm.at[pl.ds((w * RROUNDS + p) * (RW // 8),
                                              RW // 8)])
        return 0
    lax.fori_loop(0, RROUNDS, rep_round, 0)


_repack = functools.partial(
    pl.kernel,
    out_type=jax.ShapeDtypeStruct((N // 2, 8), jnp.float32),
    mesh=_mesh,
    compiler_params=_cp,
    scratch_types=[
        pltpu.VMEM((RW,), jnp.float32),
        pltpu.VMEM((RW // 8, 8), jnp.float32),
    ],
)(_repack_body)


def _sc_body(x2_hbm, vals_hbm, rows_hbm, cols_hbm, out_hbm,
             cols_v, colh_v, rows_v, rowh_v, vals_v, xg_v, ct_v, acc_sh):
    c = lax.axis_index("c")
    s = lax.axis_index("s")
    start = (c * NS + s) * PER_W

    ii = lax.iota(jnp.int32, 16)
    zf = jnp.zeros((16,), jnp.float32)
    ir2 = lax.shift_right_logical(ii, 3)
    ic8 = lax.bitwise_and(ii, 7)

    @plsc.parallel_loop(0, K // 2, unroll=8)
    def _(q):
        plsc.store_scatter(ct_v, [q * 2 + ir2, ic8], zf)

    r0 = s * ACC_PER_TILE
    pltpu.sync_copy(ct_v.at[pl.ds(0, ACC_PER_TILE)],
                    acc_sh.at[pl.ds(r0, ACC_PER_TILE)])
    plsc.subcore_barrier()

    jvs = [jnp.full((16,), j, jnp.int32) for j in range(B)]

    def chunk_body(i, _):
        base = start + i * K
        pltpu.sync_copy(cols_hbm.at[pl.ds(base, K)], cols_v)
        pltpu.sync_copy(vals_hbm.at[pl.ds(base, K)], vals_v)
        pltpu.sync_copy(rows_hbm.at[pl.ds(base, K)], rows_v)

        @plsc.parallel_loop(0, Q, unroll=8)
        def _(q):
            b16 = q * 16
            colh_v[pl.ds(b16, 16)] = lax.shift_right_logical(
                cols_v[pl.ds(b16, 16)], 1)
            rowh_v[pl.ds(b16, 16)] = lax.shift_right_logical(
                rows_v[pl.ds(b16, 16)], 1)

        pltpu.sync_copy(x2_hbm.at[colh_v], xg_v)

        @plsc.parallel_loop(0, Q, unroll=4)
        def _(q):
            b16 = q * 16
            e16 = ii + b16
            v16 = vals_v[pl.ds(b16, 16)]
            par4 = lax.shift_left(
                lax.bitwise_and(cols_v[pl.ds(b16, 16)], 1), 2)
            pr4 = lax.shift_left(
                lax.bitwise_and(rows_v[pl.ds(b16, 16)], 1), 2)
            prc4 = lax.bitwise_xor(pr4, 4)
            for j in range(B):
                xf = plsc.load_gather(xg_v, [e16, par4 + jvs[j]])
                plsc.store_scatter(ct_v, [e16, pr4 + jvs[j]], v16 * xf)
                plsc.store_scatter(ct_v, [e16, prc4 + jvs[j]], zf)

        pltpu.sync_copy(ct_v, acc_sh.at[rowh_v], add=True)
        return 0

    lax.fori_loop(0, CHUNKS, chunk_body, 0)
    plsc.subcore_barrier()

    pltpu.sync_copy(acc_sh.at[pl.ds(r0, ACC_PER_TILE)],
                    xg_v.at[pl.ds(0, ACC_PER_TILE)])
    ih = lax.shift_right_logical(ii, 1)
    colc = lax.bitwise_and(ii, 1) * 4

    @plsc.parallel_loop(0, ACC_PER_TILE * 8 // 16, unroll=8)
    def _(k):
        row16 = 64 * (k // 32) + 8 * (k % 8) + ih
        col16 = colc + (k // 8) % 4
        v16 = plsc.load_gather(xg_v, [row16, col16])
        plsc.store_scatter(ct_v, [2 * k + ir2, ic8], v16)

    pltpu.sync_copy(ct_v.at[pl.ds(0, ACC_PER_TILE)],
                    out_hbm.at[c, pl.ds(r0, ACC_PER_TILE)])


_sc_project = functools.partial(
    pl.kernel,
    out_type=jax.ShapeDtypeStruct((NC, MH, 8), jnp.float32),
    mesh=_mesh,
    compiler_params=_cp,
    scratch_types=[
        pltpu.VMEM((K,), jnp.int32),
        pltpu.VMEM((K,), jnp.int32),
        pltpu.VMEM((K,), jnp.int32),
        pltpu.VMEM((K,), jnp.int32),
        pltpu.VMEM((K,), jnp.float32),
        pltpu.VMEM((K, 8), jnp.float32),
        pltpu.VMEM((K, 8), jnp.float32),
        pltpu.VMEM_SHARED((MH, 8), jnp.float32),
    ],
)(_sc_body)

_LANES = M * B // 128


def _combine_body(p_ref, o_ref):
    o_ref[...] = SCALE * (p_ref[0] + p_ref[1])


_combine = pl.pallas_call(
    _combine_body,
    out_shape=jax.ShapeDtypeStruct((_LANES, 128), jnp.float32),
)


@jax.jit
def kernel(x, H_vals, H_rows, H_cols):
    xp = x.T.reshape(B, N // 128, 128).transpose(1, 0, 2).reshape(N * B)
    x2 = _repack(xp)
    parts = _sc_project(x2, H_vals, H_rows, H_cols)
    g = _combine(parts.reshape(NC, _LANES, 128))
    return (g.reshape(M * B // 512, B, 128).transpose(1, 0, 2)
            .reshape(B, M).T.reshape(1, 1, M, B))


# speedup vs baseline: 201.7308x; 1.1224x over previous
import functools

import jax
import jax.numpy as jnp
from jax import lax
from jax.experimental import pallas as pl
from jax.experimental.pallas import tpu as pltpu
from jax.experimental.pallas import tpu_sc as plsc

M = 92160
N = 512 ** 2
B = 4
NNZ = M * 32
SCALE = 0.82 * 0.0322

NC = 2
NS = 16
NW = NC * NS
PER_W = NNZ // NW
K = 3840
CHUNKS = PER_W // K
Q = K // 16
MH = M // 2
ACC_PER_TILE = MH // NS

RW = 8192
RROUNDS = (N * B) // (NW * RW)
RG = RW // 16

_mesh = plsc.VectorSubcoreMesh(core_axis_name="c", subcore_axis_name="s")
_cp = pltpu.CompilerParams(needs_layout_passes=False, use_tc_tiling_on_sc=False)


def _repack_body(xp_hbm, x2_hbm, xr_v, xw_v):
    c = lax.axis_index("c")
    s = lax.axis_index("s")
    w = c * NS + s

    ii = lax.iota(jnp.int32, 16)
    ir2 = lax.shift_right_logical(ii, 3)
    ic8 = lax.bitwise_and(ii, 7)
    rconst = lax.bitwise_and(ii, 3) * 128 + 2 * ir2 + \
        lax.shift_right_logical(lax.bitwise_and(ii, 7), 2)

    def rep_round(p, _):
        fb = (w * RROUNDS + p) * RW
        pltpu.sync_copy(xp_hbm.at[pl.ds(fb, RW)], xr_v)

        @plsc.parallel_loop(0, RG, unroll=8)
        def _(g):
            binidx = (g // 32) * 512 + (4 * g) % 128
            v16 = plsc.load_gather(xr_v, [rconst + binidx])
            plsc.store_scatter(xw_v, [2 * g + ir2, ic8], v16)

        pltpu.sync_copy(xw_v, x2_hbm.at[pl.ds((w * RROUNDS + p) * (RW // 8),
                                              RW // 8)])
        return 0
    lax.fori_loop(0, RROUNDS, rep_round, 0)


_repack = functools.partial(
    pl.kernel,
    out_type=jax.ShapeDtypeStruct((N // 2, 8), jnp.float32),
    mesh=_mesh,
    compiler_params=_cp,
    scratch_types=[
        pltpu.VMEM((RW,), jnp.float32),
        pltpu.VMEM((RW // 8, 8), jnp.float32),
    ],
)(_repack_body)


def _sc_body(x2_hbm, vals_hbm, rows_hbm, cols_hbm, out_hbm,
             cols0_v, cols1_v, rows0_v, rows1_v, vals0_v, vals1_v,
             colh_v, rowh_v, xg_v, ct_v, acc_sh, isem0, isem1):
    c = lax.axis_index("c")
    s = lax.axis_index("s")
    start = (c * NS + s) * PER_W

    ii = lax.iota(jnp.int32, 16)
    zf = jnp.zeros((16,), jnp.float32)
    ir2 = lax.shift_right_logical(ii, 3)
    ic8 = lax.bitwise_and(ii, 7)

    @plsc.parallel_loop(0, K // 2, unroll=8)
    def _(q):
        plsc.store_scatter(ct_v, [q * 2 + ir2, ic8], zf)

    r0 = s * ACC_PER_TILE
    pltpu.sync_copy(ct_v.at[pl.ds(0, ACC_PER_TILE)],
                    acc_sh.at[pl.ds(r0, ACC_PER_TILE)])
    plsc.subcore_barrier()

    jvs = [jnp.full((16,), j, jnp.int32) for j in range(B)]
    bufs = [(cols0_v, vals0_v, rows0_v, isem0),
            (cols1_v, vals1_v, rows1_v, isem1)]

    def fire_idx(j, cv, vv, rv, sem):
        base = start + j * K
        pltpu.async_copy(cols_hbm.at[pl.ds(base, K)], cv, sem)
        pltpu.async_copy(vals_hbm.at[pl.ds(base, K)], vv, sem)
        pltpu.async_copy(rows_hbm.at[pl.ds(base, K)], rv, sem)

    def wait_idx(j, cv, vv, rv, sem):
        base = start + j * K
        pltpu.make_async_copy(cols_hbm.at[pl.ds(base, K)], cv, sem).wait()
        pltpu.make_async_copy(vals_hbm.at[pl.ds(base, K)], vv, sem).wait()
        pltpu.make_async_copy(rows_hbm.at[pl.ds(base, K)], rv, sem).wait()

    fire_idx(0, *bufs[0])

    def chunk_pair(i, _):
        for sub in range(2):
            cv, vv, rv, sem = bufs[sub]
            j = 2 * i + sub
            wait_idx(j, cv, vv, rv, sem)

            @plsc.parallel_loop(0, Q, unroll=8)
            def _(q):
                b16 = q * 16
                colh_v[pl.ds(b16, 16)] = lax.shift_right_logical(
                    cv[pl.ds(b16, 16)], 1)
                rowh_v[pl.ds(b16, 16)] = lax.shift_right_logical(
                    rv[pl.ds(b16, 16)], 1)

            ncv, nvv, nrv, nsem = bufs[1 - sub]
            if sub == 0:
                fire_idx(j + 1, ncv, nvv, nrv, nsem)
            else:
                @pl.when(i < CHUNKS // 2 - 1)
                def _():
                    fire_idx(j + 1, ncv, nvv, nrv, nsem)
            pltpu.sync_copy(x2_hbm.at[colh_v], xg_v)

            @plsc.parallel_loop(0, Q, unroll=4)
            def _(q):
                b16 = q * 16
                e16 = ii + b16
                v16 = vv[pl.ds(b16, 16)]
                par4 = lax.shift_left(
                    lax.bitwise_and(cv[pl.ds(b16, 16)], 1), 2)
                pr4 = lax.shift_left(
                    lax.bitwise_and(rv[pl.ds(b16, 16)], 1), 2)
                prc4 = lax.bitwise_xor(pr4, 4)
                for j2 in range(B):
                    xf = plsc.load_gather(xg_v, [e16, par4 + jvs[j2]])
                    plsc.store_scatter(ct_v, [e16, pr4 + jvs[j2]], v16 * xf)
                    plsc.store_scatter(ct_v, [e16, prc4 + jvs[j2]], zf)

            pltpu.sync_copy(ct_v, acc_sh.at[rowh_v], add=True)
        return 0

    lax.fori_loop(0, CHUNKS // 2, chunk_pair, 0)
    plsc.subcore_barrier()

    pltpu.sync_copy(acc_sh.at[pl.ds(r0, ACC_PER_TILE)],
                    xg_v.at[pl.ds(0, ACC_PER_TILE)])
    ih = lax.shift_right_logical(ii, 1)
    colc = lax.bitwise_and(ii, 1) * 4

    @plsc.parallel_loop(0, ACC_PER_TILE * 8 // 16, unroll=8)
    def _(k):
        row16 = 64 * (k // 32) + 8 * (k % 8) + ih
        col16 = colc + (k // 8) % 4
        v16 = plsc.load_gather(xg_v, [row16, col16])
        plsc.store_scatter(ct_v, [2 * k + ir2, ic8], v16)

    pltpu.sync_copy(ct_v.at[pl.ds(0, ACC_PER_TILE)],
                    out_hbm.at[c, pl.ds(r0, ACC_PER_TILE)])


_sc_project = functools.partial(
    pl.kernel,
    out_type=jax.ShapeDtypeStruct((NC, MH, 8), jnp.float32),
    mesh=_mesh,
    compiler_params=_cp,
    scratch_types=[
        pltpu.VMEM((K,), jnp.int32),
        pltpu.VMEM((K,), jnp.int32),
        pltpu.VMEM((K,), jnp.int32),
        pltpu.VMEM((K,), jnp.int32),
        pltpu.VMEM((K,), jnp.float32),
        pltpu.VMEM((K,), jnp.float32),
        pltpu.VMEM((K,), jnp.int32),
        pltpu.VMEM((K,), jnp.int32),
        pltpu.VMEM((K, 8), jnp.float32),
        pltpu.VMEM((K, 8), jnp.float32),
        pltpu.VMEM_SHARED((MH, 8), jnp.float32),
        pltpu.SemaphoreType.DMA,
        pltpu.SemaphoreType.DMA,
    ],
)(_sc_body)

_LANES = M * B // 128


def _combine_body(p_ref, o_ref):
    o_ref[...] = SCALE * (p_ref[0] + p_ref[1])


_combine = pl.pallas_call(
    _combine_body,
    out_shape=jax.ShapeDtypeStruct((_LANES, 128), jnp.float32),
)


@jax.jit
def kernel(x, H_vals, H_rows, H_cols):
    xp = x.T.reshape(B, N // 128, 128).transpose(1, 0, 2).reshape(N * B)
    x2 = _repack(xp)
    parts = _sc_project(x2, H_vals, H_rows, H_cols)
    g = _combine(parts.reshape(NC, _LANES, 128))
    return (g.reshape(M * B // 512, B, 128).transpose(1, 0, 2)
            .reshape(B, M).T.reshape(1, 1, M, B))
